# baseline (speedup 1.0000x reference)
import jax
import jax.numpy as jnp
from jax import lax
from jax.experimental import pallas as pl
from jax.experimental.pallas import tpu as pltpu
from jax.experimental.pallas import tpu_sc as plsc

_L = 16
_NC = 1
_NS = 16
_NW = _NC * _NS
_N = 1024
_CPW = _N // _NW
_ROWS_X = 2
_COLS_X = 3
_NNZ = 4


def _body(xind_hbm, xval_hbm, y_hbm, out_hbm,
          xind_v, xval_v, y_v, out_v, sem_c, sem_y):
    wid = lax.axis_index("s") * _NC + lax.axis_index("c")
    base = wid * _CPW

    coo_copies = [
        pltpu.async_copy(xind_hbm.at[0], xind_v.at[0, pl.ds(0, _NNZ)], sem_c),
        pltpu.async_copy(xind_hbm.at[1], xind_v.at[1, pl.ds(0, _NNZ)], sem_c),
        pltpu.async_copy(xval_hbm, xval_v.at[pl.ds(0, _NNZ)], sem_c),
    ]
    y_copies = [
        pltpu.async_copy(y_hbm.at[j, pl.ds(base, _CPW)], y_v.at[j], sem_y)
        for j in range(_COLS_X)
    ]
    for c in coo_copies:
        c.wait()

    row = xind_v[0, :]
    col = xind_v[1, :]
    val = xval_v[...]
    coeff = [[jnp.float32(0.0)] * _COLS_X for _ in range(_ROWS_X)]
    for k in range(_NNZ):
        rk = row[k]
        ck = col[k]
        vk = val[k]
        for i in range(_ROWS_X):
            for j in range(_COLS_X):
                hit = (rk == i) & (ck == j)
                coeff[i][j] = coeff[i][j] + jnp.where(hit, vk, 0.0)

    for c in y_copies:
        c.wait()

    for i in range(_ROWS_X):
        for g in range(_CPW // _L):
            sl = pl.ds(g * _L, _L)
            acc = coeff[i][0] * y_v[0, sl]
            for j in range(1, _COLS_X):
                acc = acc + coeff[i][j] * y_v[j, sl]
            out_v[i, sl] = acc

    outs = [
        pltpu.async_copy(out_v.at[i], out_hbm.at[i, pl.ds(base, _CPW)], sem_c)
        for i in range(_ROWS_X)
    ]
    for c in outs:
        c.wait()


@jax.jit
def _spmm(xind, xval, y):
    mesh = plsc.VectorSubcoreMesh(
        core_axis_name="c", subcore_axis_name="s", num_cores=_NC
    )
    return pl.kernel(
        _body,
        mesh=mesh,
        out_type=jax.ShapeDtypeStruct((_ROWS_X, _N), jnp.float32),
        scratch_types=[
            pltpu.VMEM((2, _L), jnp.int32),
            pltpu.VMEM((_L,), jnp.float32),
            pltpu.VMEM((_COLS_X, _CPW), jnp.float32),
            pltpu.VMEM((_ROWS_X, _CPW), jnp.float32),
            pltpu.SemaphoreType.DMA,
            pltpu.SemaphoreType.DMA,
        ],
    )(xind, xval, y)


def kernel(xind, xval, y):
    return _spmm(xind, xval, y)


# speedup vs baseline: 1.0021x; 1.0021x over previous
import jax
import jax.numpy as jnp
from jax import lax
from jax.experimental import pallas as pl
from jax.experimental.pallas import tpu as pltpu
from jax.experimental.pallas import tpu_sc as plsc

_L = 16
_NC = 1
_NS = 16
_NW = _NC * _NS
_N = 1024
_CPW = _N // _NW
_ROWS_X = 2
_COLS_X = 3
_NNZ = 4


def _body(xind_hbm, xval_hbm, y_hbm, out_hbm,
          xind_v, xval_v, y_v, out_v, sem_c, sem_y):
    wid = lax.axis_index("s") * _NC + lax.axis_index("c")
    base = wid * _CPW

    coo_copies = [
        pltpu.async_copy(xind_hbm.at[0], xind_v.at[0, pl.ds(0, _NNZ)], sem_c),
        pltpu.async_copy(xind_hbm.at[1], xind_v.at[1, pl.ds(0, _NNZ)], sem_c),
        pltpu.async_copy(xval_hbm, xval_v.at[pl.ds(0, _NNZ)], sem_c),
    ]
    y_copies = [
        pltpu.async_copy(y_hbm.at[j, pl.ds(base, _CPW)], y_v.at[j], sem_y)
        for j in range(_COLS_X)
    ]
    for c in coo_copies:
        c.wait()

    row = xind_v[0, :]
    col = xind_v[1, :]
    val = xval_v[...]
    coeff = [[jnp.float32(0.0)] * _COLS_X for _ in range(_ROWS_X)]
    for k in range(_NNZ):
        rk = row[k]
        ck = col[k]
        vk = val[k]
        for i in range(_ROWS_X):
            for j in range(_COLS_X):
                hit = (rk == i) & (ck == j)
                coeff[i][j] = coeff[i][j] + jnp.where(hit, vk, 0.0)

    for c in y_copies:
        c.wait()

    outs = []
    for i in range(_ROWS_X):
        for g in range(_CPW // _L):
            sl = pl.ds(g * _L, _L)
            acc = coeff[i][0] * y_v[0, sl]
            for j in range(1, _COLS_X):
                acc = acc + coeff[i][j] * y_v[j, sl]
            out_v[i, sl] = acc
        outs.append(
            pltpu.async_copy(out_v.at[i], out_hbm.at[i, pl.ds(base, _CPW)], sem_c)
        )
    for c in outs:
        c.wait()


@jax.jit
def _spmm(xind, xval, y):
    mesh = plsc.VectorSubcoreMesh(
        core_axis_name="c", subcore_axis_name="s", num_cores=_NC
    )
    return pl.kernel(
        _body,
        mesh=mesh,
        out_type=jax.ShapeDtypeStruct((_ROWS_X, _N), jnp.float32),
        scratch_types=[
            pltpu.VMEM((2, _L), jnp.int32),
            pltpu.VMEM((_L,), jnp.float32),
            pltpu.VMEM((_COLS_X, _CPW), jnp.float32),
            pltpu.VMEM((_ROWS_X, _CPW), jnp.float32),
            pltpu.SemaphoreType.DMA,
            pltpu.SemaphoreType.DMA,
        ],
    )(xind, xval, y)


def kernel(xind, xval, y):
    return _spmm(xind, xval, y)
